# Initial kernel scaffold; baseline (speedup 1.0000x reference)
import jax
import jax.numpy as jnp
from jax.experimental import pallas as pl


def kernel(hidden_states, adapter_ids, W, b, lora_a, lora_b):
    raise NotImplementedError("write your pallas kernel here")


# speedup vs baseline: 1.0708x; 1.0708x over previous
import jax
import jax.numpy as jnp
from jax.experimental import pallas as pl
from jax.experimental.pallas import tpu as pltpu

_B, _S, _D_IN, _D_OUT, _E, _R = 4, 2048, 2048, 2048, 8, 8
_SCALING = 16.0 / 8.0
_BS = 512


def _fused_body(ids_ref, x_ref, w_ref, bias_ref, a_ref, bl_ref, o_ref):
    x = x_ref[0].astype(jnp.bfloat16)
    w = w_ref[...].astype(jnp.bfloat16)
    acc = jnp.dot(x, w, preferred_element_type=jnp.float32)
    a = a_ref[0].astype(jnp.bfloat16)
    lr = jnp.dot(x, a, preferred_element_type=jnp.float32)
    bl = bl_ref[0].astype(jnp.bfloat16)
    delta = jnp.dot((_SCALING * lr).astype(jnp.bfloat16), bl,
                    preferred_element_type=jnp.float32)
    o_ref[0] = acc + delta + bias_ref[...]


def kernel(hidden_states, adapter_ids, W, b, lora_a, lora_b):
    ids = adapter_ids.astype(jnp.int32)
    bias2d = b.reshape(1, _D_OUT)
    grid = (_B, _S // _BS)
    grid_spec = pltpu.PrefetchScalarGridSpec(
        num_scalar_prefetch=1,
        grid=grid,
        in_specs=[
            pl.BlockSpec((1, _BS, _D_IN), lambda bi, si, ids_ref: (bi, si, 0)),
            pl.BlockSpec((_D_IN, _D_OUT), lambda bi, si, ids_ref: (0, 0)),
            pl.BlockSpec((1, _D_OUT), lambda bi, si, ids_ref: (0, 0)),
            pl.BlockSpec((1, _D_IN, _R),
                         lambda bi, si, ids_ref: (ids_ref[bi], 0, 0)),
            pl.BlockSpec((1, _R, _D_OUT),
                         lambda bi, si, ids_ref: (ids_ref[bi], 0, 0)),
        ],
        out_specs=pl.BlockSpec((1, _BS, _D_OUT),
                               lambda bi, si, ids_ref: (bi, si, 0)),
    )
    out = pl.pallas_call(
        _fused_body,
        grid_spec=grid_spec,
        out_shape=jax.ShapeDtypeStruct((_B, _S, _D_OUT), jnp.float32),
    )(ids, hidden_states, W, bias2d, lora_a, lora_b)
    return out
